# baseline (speedup 1.0000x reference)
import functools

import jax
import jax.numpy as jnp
from jax import lax
from jax.experimental import pallas as pl
from jax.experimental.pallas import tpu as pltpu
from jax.experimental.pallas import tpu_sc as plsc

D = 64
B = 16384
TW = 128

_INFO = plsc.get_sparse_core_info()
NC = _INFO.num_cores
NS = _INFO.num_subcores
L = _INFO.num_lanes
NW = NC * NS
BPW = B // NW
NGROUP = BPW // L
NBUF = 4
AHEAD = NBUF - 1

_MAGIC = 0x5F3759DF
_INV_EPS = 1e8


def _rsqrt16(x):
    i = plsc.bitcast(x, jnp.int32)
    i = jnp.full((L,), _MAGIC, jnp.int32) - (i >> 1)
    y = plsc.bitcast(i, jnp.float32)
    half_x = x * 0.5
    for _ in range(3):
        y = y * (1.5 - half_x * y * y)
    return jnp.minimum(y, jnp.full((L,), _INV_EPS, jnp.float32))


def _sc_body(uidx_hbm, iidx_hbm, utab_hbm, itab_hbm, out_hbm,
             uidx_v, iidx_v, ubuf_v, ibuf_v, out_v, sem):
    wid = lax.axis_index("s") * NC + lax.axis_index("c")
    base = wid * BPW

    pltpu.sync_copy(uidx_hbm.at[pl.ds(base, BPW)], uidx_v)
    pltpu.sync_copy(iidx_hbm.at[pl.ds(base, BPW)], iidx_v)

    def fire(ur, ir, slot):
        pltpu.async_copy(utab_hbm.at[:, pl.ds((ur >> 7) * TW, TW)],
                         ubuf_v.at[slot], sem)
        pltpu.async_copy(itab_hbm.at[:, pl.ds((ir >> 7) * TW, TW)],
                         ibuf_v.at[slot], sem)

    def drain(ur, ir, slot):
        pltpu.make_async_copy(utab_hbm.at[:, pl.ds((ur >> 7) * TW, TW)],
                              ubuf_v.at[slot], sem).wait()
        pltpu.make_async_copy(itab_hbm.at[:, pl.ds((ir >> 7) * TW, TW)],
                              ibuf_v.at[slot], sem).wait()

    def group_vecs(g):
        return uidx_v[pl.ds(g * L, L)], iidx_v[pl.ds(g * L, L)]

    uvec0, ivec0 = group_vecs(0)
    for j in range(AHEAD):
        fire(uvec0[j], ivec0[j], j)

    lane = lax.iota(jnp.int32, L)
    ones = jnp.full((L,), 1, jnp.int32)

    def step(g, carry):
        uvec, ivec = group_vecs(g)
        uvec_n, ivec_n = group_vecs(lax.rem(g + 1, NGROUP))
        dacc = jnp.zeros((L,), jnp.float32)
        uacc = jnp.zeros((L,), jnp.float32)
        vacc = jnp.zeros((L,), jnp.float32)
        for j in range(L):
            slot = (j + AHEAD) % NBUF

            if j + AHEAD < L:
                fire(uvec[j + AHEAD], ivec[j + AHEAD], slot)
            else:
                @pl.when(g + 1 < NGROUP)
                def _pref():
                    fire(uvec_n[j + AHEAD - L], ivec_n[j + AHEAD - L], slot)

            drain(uvec[j], ivec[j], j % NBUF)
            cu = ones * (uvec[j] & (TW - 1))
            ci = ones * (ivec[j] & (TW - 1))
            ubuf = ubuf_v.at[j % NBUF]
            ibuf = ibuf_v.at[j % NBUF]
            dot = jnp.zeros((L,), jnp.float32)
            uu = jnp.zeros((L,), jnp.float32)
            vv = jnp.zeros((L,), jnp.float32)
            for q in range(D // L):
                kv = lane + q * L
                u = plsc.load_gather(ubuf, [kv, cu])
                v = plsc.load_gather(ibuf, [kv, ci])
                dot = dot + u * v
                uu = uu + u * u
                vv = vv + v * v
            sel = lane == j
            dacc = jnp.where(sel, plsc.cumsum(dot)[L - 1], dacc)
            uacc = jnp.where(sel, plsc.cumsum(uu)[L - 1], uacc)
            vacc = jnp.where(sel, plsc.cumsum(vv)[L - 1], vacc)
        res = dacc * _rsqrt16(uacc) * _rsqrt16(vacc)
        out_v[pl.ds(g * L, L)] = res
        return carry

    lax.fori_loop(0, NGROUP, step, 0)

    pltpu.sync_copy(out_v, out_hbm.at[pl.ds(base, BPW)])


def _unperm_body(res_hbm, rank_hbm, out_hbm, res_v, rank_v, out_v):
    wid = lax.axis_index("s") * NC + lax.axis_index("c")
    base = wid * BPW
    pltpu.sync_copy(res_hbm, res_v)
    pltpu.sync_copy(rank_hbm.at[pl.ds(base, BPW)], rank_v)
    for g in range(NGROUP):
        rk = rank_v[pl.ds(g * L, L)]
        out_v[pl.ds(g * L, L)] = plsc.load_gather(res_v, [rk])
    pltpu.sync_copy(out_v, out_hbm.at[pl.ds(base, BPW)])


def kernel(user_indices, item_indices, user_table, item_table):
    mesh = plsc.VectorSubcoreMesh(core_axis_name="c", subcore_axis_name="s")
    params = pltpu.CompilerParams(needs_layout_passes=False)
    gather_k = functools.partial(
        pl.kernel,
        mesh=mesh,
        out_type=jax.ShapeDtypeStruct((B,), jnp.float32),
        compiler_params=params,
        scratch_types=[
            pltpu.VMEM((BPW,), jnp.int32),
            pltpu.VMEM((BPW,), jnp.int32),
            pltpu.VMEM((NBUF, D, TW), jnp.float32),
            pltpu.VMEM((NBUF, D, TW), jnp.float32),
            pltpu.VMEM((BPW,), jnp.float32),
            pltpu.SemaphoreType.DMA,
        ],
    )(_sc_body)
    unperm_k = functools.partial(
        pl.kernel,
        mesh=mesh,
        out_type=jax.ShapeDtypeStruct((B,), jnp.float32),
        compiler_params=params,
        scratch_types=[
            pltpu.VMEM((B,), jnp.float32),
            pltpu.VMEM((BPW,), jnp.int32),
            pltpu.VMEM((BPW,), jnp.float32),
        ],
    )(_unperm_body)

    ui = user_indices.astype(jnp.int32)
    ii = item_indices.astype(jnp.int32)
    pos = jnp.arange(B, dtype=jnp.int32)
    su, perm = lax.sort((ui, pos), num_keys=1)
    ii_p = jnp.take(ii, perm)
    rank = jnp.zeros((B,), jnp.int32).at[perm].set(pos)
    res_sorted = gather_k(su, ii_p, user_table.T, item_table.T)
    return unperm_k(res_sorted, rank)


# speedup vs baseline: 1.3055x; 1.3055x over previous
import functools

import jax
import jax.numpy as jnp
from jax import lax
from jax.experimental import pallas as pl
from jax.experimental.pallas import tpu as pltpu
from jax.experimental.pallas import tpu_sc as plsc

D = 64
B = 16384
TW = 128

_INFO = plsc.get_sparse_core_info()
NC = _INFO.num_cores
NS = _INFO.num_subcores
L = _INFO.num_lanes
NW = NC * NS
BPW = B // NW
NGROUP = BPW // L
NBUF = 4
AHEAD = NBUF - 1

_MAGIC = 0x5F3759DF
_INV_EPS = 1e8


def _rsqrt16(x):
    i = plsc.bitcast(x, jnp.int32)
    i = jnp.full((L,), _MAGIC, jnp.int32) - (i >> 1)
    y = plsc.bitcast(i, jnp.float32)
    half_x = x * 0.5
    for _ in range(3):
        y = y * (1.5 - half_x * y * y)
    return jnp.minimum(y, jnp.full((L,), _INV_EPS, jnp.float32))


def _sc_body(uidx_hbm, iidx_hbm, utab_hbm, itab_hbm, out_hbm,
             uidx_v, iidx_v, ubuf_v, ibuf_v, out_v, usem, isem):
    wid = lax.axis_index("s") * NC + lax.axis_index("c")
    base = wid * BPW

    pltpu.sync_copy(uidx_hbm.at[pl.ds(base, BPW)], uidx_v)
    pltpu.sync_copy(iidx_hbm.at[pl.ds(base, BPW)], iidx_v)

    def ufire(ut, slot):
        pltpu.async_copy(utab_hbm.at[:, pl.ds(ut * TW, TW)],
                         ubuf_v.at[slot], usem)

    def udrain(ut, slot):
        pltpu.make_async_copy(utab_hbm.at[:, pl.ds(ut * TW, TW)],
                              ubuf_v.at[slot], usem).wait()

    def ifire(ir, slot):
        pltpu.async_copy(itab_hbm.at[:, pl.ds((ir >> 7) * TW, TW)],
                         ibuf_v.at[slot], isem)

    def idrain(ir, slot):
        pltpu.make_async_copy(itab_hbm.at[:, pl.ds((ir >> 7) * TW, TW)],
                              ibuf_v.at[slot], isem).wait()

    def group_vecs(g):
        return uidx_v[pl.ds(g * L, L)], iidx_v[pl.ds(g * L, L)]

    uvec0, ivec0 = group_vecs(0)
    ufire(uvec0[0] >> 7, 0)
    ucnt = jnp.int32(1)
    for r in range(1, AHEAD):
        f = (uvec0[r] >> 7) != (uvec0[r - 1] >> 7)
        slot = lax.rem(ucnt, NBUF)

        @pl.when(f)
        def _pf():
            ufire(uvec0[r] >> 7, slot)

        ucnt = ucnt + f.astype(jnp.int32)
    for j in range(AHEAD):
        ifire(ivec0[j], j)

    lane = lax.iota(jnp.int32, L)
    ones = jnp.full((L,), 1, jnp.int32)

    def step(g, carry):
        ucnt, ccnt = carry
        uvec, ivec = group_vecs(g)
        gn = lax.rem(g + 1, NGROUP)
        uvec_n, ivec_n = group_vecs(gn)
        pg = lax.max(g - 1, 0)
        prev_last = uidx_v[pl.ds(pg * L, L)][L - 1]
        dacc = jnp.zeros((L,), jnp.float32)
        uacc = jnp.zeros((L,), jnp.float32)
        vacc = jnp.zeros((L,), jnp.float32)
        for j in range(L):
            slot = (j + AHEAD) % NBUF

            if j + AHEAD < L:
                tu, tp = uvec[j + AHEAD] >> 7, uvec[j + AHEAD - 1] >> 7
                fp = tu != tp
                uslot = lax.rem(ucnt, NBUF)

                @pl.when(fp)
                def _pfu():
                    ufire(tu, uslot)

                ucnt = ucnt + fp.astype(jnp.int32)
                ifire(ivec[j + AHEAD], slot)
            else:
                jp = j + AHEAD - L
                tu = uvec_n[jp] >> 7
                tp = (uvec_n[jp - 1] if jp > 0 else uvec[L - 1]) >> 7
                fp = jnp.logical_and(tu != tp, g + 1 < NGROUP)
                uslot = lax.rem(ucnt, NBUF)

                @pl.when(fp)
                def _pfu2():
                    ufire(tu, uslot)

                ucnt = ucnt + fp.astype(jnp.int32)

                @pl.when(g + 1 < NGROUP)
                def _pfi():
                    ifire(ivec_n[jp], slot)

            tu_j = uvec[j] >> 7
            tp_j = (uvec[j - 1] if j > 0 else prev_last) >> 7
            cj = jnp.logical_or(tu_j != tp_j,
                                jnp.logical_and(g == 0, j == 0))
            ccnt = ccnt + cj.astype(jnp.int32)
            uslot_c = lax.rem(ccnt - 1, NBUF)

            @pl.when(cj)
            def _du():
                udrain(tu_j, uslot_c)

            idrain(ivec[j], j % NBUF)
            cu = ones * (uvec[j] & (TW - 1))
            ci = ones * (ivec[j] & (TW - 1))
            ubuf = ubuf_v.at[uslot_c]
            ibuf = ibuf_v.at[j % NBUF]
            dot = jnp.zeros((L,), jnp.float32)
            uu = jnp.zeros((L,), jnp.float32)
            vv = jnp.zeros((L,), jnp.float32)
            for q in range(D // L):
                kv = lane + q * L
                u = plsc.load_gather(ubuf, [kv, cu])
                v = plsc.load_gather(ibuf, [kv, ci])
                dot = dot + u * v
                uu = uu + u * u
                vv = vv + v * v
            sel = lane == j
            dacc = jnp.where(sel, plsc.cumsum(dot)[L - 1], dacc)
            uacc = jnp.where(sel, plsc.cumsum(uu)[L - 1], uacc)
            vacc = jnp.where(sel, plsc.cumsum(vv)[L - 1], vacc)
        res = dacc * _rsqrt16(uacc) * _rsqrt16(vacc)
        out_v[pl.ds(g * L, L)] = res
        return ucnt, ccnt

    lax.fori_loop(0, NGROUP, step, (ucnt, jnp.int32(0)))

    pltpu.sync_copy(out_v, out_hbm.at[pl.ds(base, BPW)])


def _unperm_body(res_hbm, rank_hbm, out_hbm, res_v, rank_v, out_v):
    wid = lax.axis_index("s") * NC + lax.axis_index("c")
    base = wid * BPW
    pltpu.sync_copy(res_hbm, res_v)
    pltpu.sync_copy(rank_hbm.at[pl.ds(base, BPW)], rank_v)
    for g in range(NGROUP):
        rk = rank_v[pl.ds(g * L, L)]
        out_v[pl.ds(g * L, L)] = plsc.load_gather(res_v, [rk])
    pltpu.sync_copy(out_v, out_hbm.at[pl.ds(base, BPW)])


def kernel(user_indices, item_indices, user_table, item_table):
    mesh = plsc.VectorSubcoreMesh(core_axis_name="c", subcore_axis_name="s")
    params = pltpu.CompilerParams(needs_layout_passes=False)
    gather_k = functools.partial(
        pl.kernel,
        mesh=mesh,
        out_type=jax.ShapeDtypeStruct((B,), jnp.float32),
        compiler_params=params,
        scratch_types=[
            pltpu.VMEM((BPW,), jnp.int32),
            pltpu.VMEM((BPW,), jnp.int32),
            pltpu.VMEM((NBUF, D, TW), jnp.float32),
            pltpu.VMEM((NBUF, D, TW), jnp.float32),
            pltpu.VMEM((BPW,), jnp.float32),
            pltpu.SemaphoreType.DMA,
            pltpu.SemaphoreType.DMA,
        ],
    )(_sc_body)
    unperm_k = functools.partial(
        pl.kernel,
        mesh=mesh,
        out_type=jax.ShapeDtypeStruct((B,), jnp.float32),
        compiler_params=params,
        scratch_types=[
            pltpu.VMEM((B,), jnp.float32),
            pltpu.VMEM((BPW,), jnp.int32),
            pltpu.VMEM((BPW,), jnp.float32),
        ],
    )(_unperm_body)

    ui = user_indices.astype(jnp.int32)
    ii = item_indices.astype(jnp.int32)
    pos = jnp.arange(B, dtype=jnp.int32)
    su, perm = lax.sort((ui, pos), num_keys=1)
    ii_p = jnp.take(ii, perm)
    rank = jnp.zeros((B,), jnp.int32).at[perm].set(pos)
    res_sorted = gather_k(su, ii_p, user_table.T, item_table.T)
    return unperm_k(res_sorted, rank)
